# baseline (speedup 1.0000x reference)
import functools

import jax
import jax.numpy as jnp
from jax import lax
from jax.experimental import pallas as pl
from jax.experimental.pallas import tpu as pltpu
from jax.experimental.pallas import tpu_sc as plsc

B = 1024
S = 51
IN = 512
N = B * S
E = 200000

_NC = 2
_NS = 16
_NW = _NC * _NS
_EPW = 6272
_EPAD = _NW * _EPW
_GC = 128

_T1 = 512
_T2 = 3136
_T3B = 8


def _lrelu(v):
    return jnp.where(v >= 0, v, 0.01 * v)


def _tc1_body(xb, pb, m2w1, m2b1, m2w2, m2b2, m2w3, m2b3,
              m3w1, m3b1, m3w2, m3b2, m3w3, m3b3,
              encw, encb, ew1, wn, ws,
              u_o, un_o, zn_o, zs_o):
    f32 = jnp.float32
    h = jax.nn.relu(jnp.dot(xb[...], m2w1[...], preferred_element_type=f32) + m2b1[...])
    h = jax.nn.relu(jnp.dot(h, m2w2[...], preferred_element_type=f32) + m2b2[...])
    h = jax.nn.relu(jnp.dot(h, m2w3[...], preferred_element_type=f32) + m2b3[...])
    p = jax.nn.relu(jnp.dot(pb[...], m3w1[...], preferred_element_type=f32) + m3b1[...])
    p = jax.nn.relu(jnp.dot(p, m3w2[...], preferred_element_type=f32) + m3b2[...])
    p = jax.nn.relu(jnp.dot(p, m3w3[...], preferred_element_type=f32) + m3b3[...])
    ew = encw[...]
    z = jax.nn.relu(jnp.dot(h, ew[0:64, :], preferred_element_type=f32)
                    + jnp.dot(p, ew[64:128, :], preferred_element_type=f32)
                    + encb[...])
    u = jnp.dot(z, ew1[...], preferred_element_type=f32)
    u_o[...] = u
    un_o[...] = -u
    zn_o[...] = jnp.dot(z, wn[...], preferred_element_type=f32)
    zs_o[...] = jnp.dot(z, ws[...], preferred_element_type=f32)


def _tc1(x2d, p2d, prm):
    grid = (N // _T1,)
    row = lambda i: (i, 0)
    full = lambda i: (0, 0)

    def wspec(a):
        return pl.BlockSpec(a.shape, full)

    weights = [prm['m2_w1'], prm['m2_b1'], prm['m2_w2'], prm['m2_b2'],
               prm['m2_w3'], prm['m2_b3'],
               prm['m3_w1'], prm['m3_b1'], prm['m3_w2'], prm['m3_b2'],
               prm['m3_w3'], prm['m3_b3'],
               prm['enc_w'], prm['enc_b'], prm['e_w1'],
               prm['sage_wn'], prm['sage_ws']]
    return pl.pallas_call(
        _tc1_body,
        grid=grid,
        in_specs=[pl.BlockSpec((_T1, IN), row), pl.BlockSpec((_T1, 7), row)]
                 + [wspec(w) for w in weights],
        out_specs=[pl.BlockSpec((_T1, 256), row), pl.BlockSpec((_T1, 256), row),
                   pl.BlockSpec((_T1, 128), row), pl.BlockSpec((_T1, 128), row)],
        out_shape=[jax.ShapeDtypeStruct((N, 256), jnp.float32),
                   jax.ShapeDtypeStruct((N, 256), jnp.float32),
                   jax.ShapeDtypeStruct((N, 128), jnp.float32),
                   jax.ShapeDtypeStruct((N, 128), jnp.float32)],
    )(x2d, p2d, *weights)


def _g1_body(u_hbm, un_hbm, zn_hbm, src_hbm, dst_hbm, diff_hbm, zsrc_hbm,
             sidx, didx, rs, rd, rz, sem1, sem2, sem3):
    wid = lax.axis_index("s") * _NC + lax.axis_index("c")
    base = wid * _EPW
    pltpu.sync_copy(src_hbm.at[pl.ds(base, _EPW)], sidx)
    pltpu.sync_copy(dst_hbm.at[pl.ds(base, _EPW)], didx)

    def chunk(k, carry):
        off = k * _GC
        cp1 = pltpu.async_copy(u_hbm.at[sidx.at[pl.ds(off, _GC)]], rs, sem1)
        cp2 = pltpu.async_copy(un_hbm.at[didx.at[pl.ds(off, _GC)]], rd, sem2)
        cp3 = pltpu.async_copy(zn_hbm.at[sidx.at[pl.ds(off, _GC)]], rz, sem3)
        cp1.wait()
        cp2.wait()

        def row(r, c2):
            for j in range(16):
                sl = pl.ds(j * 16, 16)
                rs[r, sl] = rs[r, sl] + rd[r, sl]
            return c2

        lax.fori_loop(0, _GC, row, 0)
        pltpu.sync_copy(rs, diff_hbm.at[pl.ds(base + off, _GC)])
        cp3.wait()
        pltpu.sync_copy(rz, zsrc_hbm.at[pl.ds(base + off, _GC)])
        return carry

    lax.fori_loop(0, _EPW // _GC, chunk, 0)


def _g1(u, un, zn, src_p, dst_p):
    mesh = plsc.VectorSubcoreMesh(core_axis_name="c", subcore_axis_name="s")
    f = pl.kernel(
        _g1_body,
        out_type=[jax.ShapeDtypeStruct((_EPAD, 256), jnp.float32),
                  jax.ShapeDtypeStruct((_EPAD, 128), jnp.float32)],
        mesh=mesh,
        scratch_types=[
            pltpu.VMEM((_EPW,), jnp.int32),
            pltpu.VMEM((_EPW,), jnp.int32),
            pltpu.VMEM((_GC, 256), jnp.float32),
            pltpu.VMEM((_GC, 256), jnp.float32),
            pltpu.VMEM((_GC, 128), jnp.float32),
            pltpu.SemaphoreType.DMA,
            pltpu.SemaphoreType.DMA,
            pltpu.SemaphoreType.DMA,
        ],
    )
    return f(u, un, zn, src_p, dst_p)


_NT = 53248
_NTT = _NT // _NS
_CCH = 128
_NCH = _EPW // _CCH
_QG = 8


def _g2_body(msg_hbm, dst_hbm, rid_hbm, agg_o, deg_o,
             idxs, idxz, gbuf, ones, zbuf, acc_sp, sem1):
    f32 = jnp.float32
    cid = lax.axis_index("c")
    sid = lax.axis_index("s")
    base = (sid * _NC + cid) * _EPW
    own0 = sid * _NTT

    def fill(r, c):
        zbuf[r, pl.ds(0, 16)] = jnp.zeros((16,), f32)
        ones[r, pl.ds(0, 16)] = jnp.full((16,), 1.0, f32)
        return c

    lax.fori_loop(0, _CCH, fill, 0)

    for q in range(_QG + 1):
        def zchunk(t, c):
            pltpu.sync_copy(rid_hbm.at[pl.ds(own0 + t * _CCH, _CCH)], idxz)
            pltpu.sync_copy(zbuf, acc_sp.at[idxz])
            return c

        lax.fori_loop(0, _NTT // _CCH, zchunk, 0)
        plsc.subcore_barrier()

        if q < _QG:
            def chunk(j, c):
                pltpu.sync_copy(dst_hbm.at[pl.ds(base + j * _CCH, _CCH)], idxs)
                pltpu.sync_copy(
                    msg_hbm.at[pl.ds(q * _EPAD + base + j * _CCH, _CCH)], gbuf)
                pltpu.sync_copy(gbuf, acc_sp.at[idxs], add=True)
                return c
        else:
            def chunk(j, c):
                pltpu.sync_copy(dst_hbm.at[pl.ds(base + j * _CCH, _CCH)], idxs)
                pltpu.sync_copy(ones, acc_sp.at[idxs], add=True)
                return c

        lax.fori_loop(0, _NCH, chunk, 0)
        plsc.subcore_barrier()

        if q < _QG:
            off = (q * _NC + cid) * _NT + own0
        else:
            off = cid * _NT + own0
        out_ref = agg_o if q < _QG else deg_o

        def ochunk(t, c):
            pltpu.sync_copy(rid_hbm.at[pl.ds(own0 + t * _CCH, _CCH)], idxz)
            pltpu.sync_copy(acc_sp.at[idxz], gbuf)
            pltpu.sync_copy(gbuf, out_ref.at[pl.ds(off + t * _CCH, _CCH)])
            return c

        lax.fori_loop(0, _NTT // _CCH, ochunk, 0)


def _g2(msg2d, dst_p, rid):
    mesh = plsc.VectorSubcoreMesh(core_axis_name="c", subcore_axis_name="s")
    f = pl.kernel(
        _g2_body,
        out_type=[jax.ShapeDtypeStruct((_QG * _NC * _NT, 16), jnp.float32),
                  jax.ShapeDtypeStruct((_NC * _NT, 16), jnp.float32)],
        mesh=mesh,
        scratch_types=[
            pltpu.VMEM((_CCH,), jnp.int32),
            pltpu.VMEM((_CCH,), jnp.int32),
            pltpu.VMEM((_CCH, 16), jnp.float32),
            pltpu.VMEM((_CCH, 16), jnp.float32),
            pltpu.VMEM((_CCH, 16), jnp.float32),
            pltpu.VMEM_SHARED((_NT, 16), jnp.float32),
            pltpu.SemaphoreType.DMA,
        ],
    )
    return f(msg2d, dst_p, rid)


def _tc2_body(db, zb, b1, w2, b2, w3, b3, w4, b4, e_o, msg_o):
    f32 = jnp.float32
    t = _lrelu(db[...] + b1[...])
    t = _lrelu(jnp.dot(t, w2[...], preferred_element_type=f32) + b2[...])
    t = _lrelu(jnp.dot(t, w3[...], preferred_element_type=f32) + b3[...])
    logit = jnp.dot(t, w4[...], preferred_element_type=f32) + b4[...]
    e = jax.nn.sigmoid(logit)
    e_o[...] = e
    msg = zb[...] * e
    for q in range(_QG):
        msg_o[q, :, :] = msg[:, q * 16:(q + 1) * 16]


def _tc2(diff, zsrc, prm):
    grid = (_EPAD // _T2,)
    row = lambda i: (i, 0)
    full = lambda i: (0, 0)
    weights = [prm['e_b1'], prm['e_w2'], prm['e_b2'],
               prm['e_w3'], prm['e_b3'], prm['e_w4'], prm['e_b4']]
    return pl.pallas_call(
        _tc2_body,
        grid=grid,
        in_specs=[pl.BlockSpec((_T2, 256), row), pl.BlockSpec((_T2, 128), row)]
                 + [pl.BlockSpec(w.shape, full) for w in weights],
        out_specs=[pl.BlockSpec((_T2, 1), row),
                   pl.BlockSpec((_QG, _T2, 16), lambda i: (0, i, 0))],
        out_shape=[jax.ShapeDtypeStruct((_EPAD, 1), jnp.float32),
                   jax.ShapeDtypeStruct((_QG, _EPAD, 16), jnp.float32)],
    )(diff, zsrc, *weights)


def _tc3_body(zsb, aggb, degb, sageb, decw, decb, a_o, est_o):
    f32 = jnp.float32
    rows = _T3B * S
    agg = aggb[0] + aggb[1]
    deg = degb[0, :, 0:1] + degb[1, :, 0:1]
    neigh = agg / jnp.maximum(deg, 1.0)
    A = zsb[...] + neigh + sageb[...]
    est = jnp.dot(A, decw[...], preferred_element_type=f32) + decb[...]
    gid = lax.broadcasted_iota(jnp.int32, (_T3B, rows), 0)
    rid = lax.broadcasted_iota(jnp.int32, (_T3B, rows), 1)
    sel0 = (rid == gid * S).astype(f32)
    est_o[...] = jnp.dot(sel0, est, preferred_element_type=f32)
    Al = _lrelu(A)
    grp = (rid // S == gid).astype(f32)
    ssq = jnp.dot(grp, Al * Al, preferred_element_type=f32)
    nrm = jnp.maximum(jnp.sqrt(ssq), 1e-12)
    rownrm = jnp.dot(grp.T, nrm, preferred_element_type=f32)
    a_o[...] = Al / rownrm


def _tc3(zs, aggp, degp, prm):
    rows = _T3B * S
    grid = (N // rows,)
    row = lambda i: (i, 0)
    full = lambda i: (0, 0)
    return pl.pallas_call(
        _tc3_body,
        grid=grid,
        in_specs=[pl.BlockSpec((rows, 128), row),
                  pl.BlockSpec((2, rows, 128), lambda i: (0, i, 0)),
                  pl.BlockSpec((2, rows, 16), lambda i: (0, i, 0)),
                  pl.BlockSpec(prm['sage_b'].shape, full),
                  pl.BlockSpec(prm['dec_w'].shape, full),
                  pl.BlockSpec(prm['dec_b'].shape, full)],
        out_specs=[pl.BlockSpec((rows, 128), row),
                   pl.BlockSpec((_T3B, 7), row)],
        out_shape=[jax.ShapeDtypeStruct((N, 128), jnp.float32),
                   jax.ShapeDtypeStruct((B, 7), jnp.float32)],
    )(zs, aggp, degp, prm['sage_b'], prm['dec_w'], prm['dec_b'])


def kernel(x, x_pose, edge_index, params):
    prm = dict(params)
    for k in ['m2_b1', 'm2_b2', 'm2_b3', 'm3_b1', 'm3_b2', 'm3_b3',
              'enc_b', 'e_b1', 'e_b2', 'e_b3', 'e_b4', 'sage_b', 'dec_b']:
        prm[k] = prm[k].reshape(1, -1)
    x2d = x.reshape(N, IN)
    p2d = x_pose.reshape(N, 7)
    pad0 = jnp.zeros((_EPAD - E,), jnp.int32)
    padn = jnp.full((_EPAD - E,), N, jnp.int32)
    src_p = jnp.concatenate([edge_index[0], pad0])
    dst_g1 = jnp.concatenate([edge_index[1], pad0])
    dst_g2 = jnp.concatenate([edge_index[1], padn])

    u, un, zn, zs = _tc1(x2d, p2d, prm)
    diff, zsrc = _g1(u, un, zn, src_p, dst_g1)
    e_pad, msg3d = _tc2(diff, zsrc, prm)
    e = e_pad[:E]
    rid = jnp.arange(_NT, dtype=jnp.int32)
    aggq, degf = _g2(msg3d.reshape(_QG * _EPAD, 16), dst_g2, rid)
    degp = degf.reshape(_NC, _NT, 16)[:, :N]
    aggp = (aggq.reshape(_QG, _NC, _NT, 16)[:, :, :N]
            .transpose(1, 2, 0, 3).reshape(_NC, N, 128))
    a2d, est0 = _tc3(zs, aggp, degp, prm)

    A = a2d.reshape(B, S, 128)
    pos = est0[:, 0:3]
    ori = est0[:, 3:7]
    return (A, e, pos, ori)
